# baseline (speedup 1.0000x reference)
import functools

import jax
import jax.numpy as jnp
from jax import lax
from jax.experimental import pallas as pl
from jax.experimental.pallas import tpu as pltpu
from jax.experimental.pallas import tpu_sc as plsc

OUT = 64
B = 16384
CONST_VAL = 0.1
VU = 1000000
VI = 100000

NC = 2
NS = 16
L = 16
NW = NC * NS

W = 512
NFULL_U = VU // W
NFULL_I = VI // W
PAD_U = 999936
PAD_I = 99968
NBIN = 64
DUMP = 63
BSORT = B + NBIN * L
CBUF = 2048
NCTX = B * OUT // NW // CBUF

_mesh = plsc.VectorSubcoreMesh(core_axis_name="c", subcore_axis_name="s")


@functools.partial(
    pl.kernel,
    mesh=_mesh,
    compiler_params=pltpu.CompilerParams(
        use_tc_tiling_on_sc=True, needs_layout_passes=False),
    out_type=(
        jax.ShapeDtypeStruct((B * OUT,), jnp.float32),
        jax.ShapeDtypeStruct((B * OUT,), jnp.float32),
        jax.ShapeDtypeStruct((B * OUT,), jnp.float32),
    ),
    scratch_types=[
        pltpu.VMEM((B,), jnp.int32),
        pltpu.VMEM((BSORT,), jnp.int32),
        pltpu.VMEM((80,), jnp.int32),
        pltpu.VMEM((80,), jnp.int32),
        pltpu.VMEM((80,), jnp.int32),
        pltpu.VMEM((2, OUT, W), jnp.float32),
        pltpu.VMEM((2 * L * OUT,), jnp.float32),
        pltpu.VMEM((L * OUT,), jnp.int32),
        pltpu.VMEM((CBUF,), jnp.float32),
        pltpu.SemaphoreType.DMA,
        pltpu.SemaphoreType.DMA,
        pltpu.SemaphoreType.DMA,
    ],
)
def _sc_encode(uid_hbm, iid_hbm, wut_hbm, wit_hbm, upad_hbm, ipad_hbm,
               ou_hbm, oi_hbm, oc_hbm,
               ids_v, bsort_v, hist_v, seg_v, nxt_v, buf3, ring_v, drain_v,
               cb_v, sem, sem_w, sem_c):
    wid = lax.axis_index("s") * NC + lax.axis_index("c")
    lanes = lax.iota(jnp.int32, L)
    ones = jnp.ones((L,), jnp.int32)
    zeros = jnp.zeros((L,), jnp.int32)

    cvec = jnp.full((L,), CONST_VAL, jnp.float32)

    def cfill(i, _):
        cb_v[pl.ds(i * L, L)] = cvec
        return ()

    lax.fori_loop(0, CBUF // L, cfill, (), unroll=8)
    base_c = wid * (B * OUT // NW)
    for j in range(NCTX):
        pltpu.async_copy(cb_v, oc_hbm.at[pl.ds(base_c + j * CBUF, CBUF)],
                         sem_c)

    def run_table(idx_hbm, wt_hbm, pad_hbm, out_hbm, nfull, item_station):
        ntot = nfull + 1
        n_me = (ntot + (NW - 1) - wid) // NW

        pltpu.sync_copy(idx_hbm, ids_v)

        @pl.when(n_me > 0)
        def _prologue():
            pltpu.async_copy(wt_hbm.at[:, pl.ds(wid * W, W)], buf3.at[0],
                             sem_w)

        for c in range(5):
            hist_v[pl.ds(c * L, L)] = zeros

        def p1(i, _):
            idv = ids_v[pl.ds(i * L, L)]
            g = idv // W
            lgv = jnp.where((g & (NW - 1)) == wid, g >> 5, DUMP)
            plsc.addupdate_scatter(hist_v, [lgv], ones)
            return ()

        lax.fori_loop(0, B // L, p1, (), unroll=4)

        carry = jnp.int32(0)
        for c in range(5):
            h = hist_v[pl.ds(c * L, L)]
            hp = (h + (L - 1)) & ~(L - 1)
            ps = plsc.cumsum(hp)
            seg = ps - hp + carry
            seg_v[pl.ds(c * L, L)] = seg
            nxt_v[pl.ds(c * L, L)] = seg
            carry = carry + ps[L - 1]

        def p3(i, _):
            idv = ids_v[pl.ds(i * L, L)]
            bv = lanes + i * L
            g = idv // W
            lgv = jnp.where((g & (NW - 1)) == wid, g >> 5, DUMP)
            nxtv = plsc.load_gather(nxt_v, [lgv])
            rank, last = plsc.scan_count(lgv)
            plsc.store_scatter(bsort_v, [nxtv + rank - 1], bv)
            plsc.store_scatter(nxt_v, [jnp.where(last, lgv, NBIN)],
                               nxtv + rank)
            return ()

        lax.fori_loop(0, B // L, p3, (), unroll=4)

        cofs = [jnp.broadcast_to(jnp.int32(c0), (L,)) + lanes
                for c0 in range(0, OUT, L)]

        def drain16():
            pltpu.make_async_copy(idx_hbm.at[pl.ds(0, L * OUT)], drain_v,
                                  sem).wait()

        def process_segment(par, base, lg, gidx):
            parv = jnp.broadcast_to(par, (L,))
            cb = (lg >> 4) << 4
            lane = lg & (L - 1)
            hch = hist_v[pl.ds(cb, L)]
            sch = seg_v[pl.ds(cb, L)]
            kw = jnp.max(jnp.where(lanes == lane, hch, 0))
            s = jnp.max(jnp.where(lanes == lane, sch, 0))

            def grp(t16, gi):
                bv0 = bsort_v[pl.ds(s + t16 * L, L)]
                rem = kw - t16 * L
                bvc = jnp.where(lanes < rem, bv0, 0)
                colv = jnp.where(lanes < rem,
                                 plsc.load_gather(ids_v, [bvc]) - base, 0)
                bank = (gi & 1) * (L * OUT)

                @pl.when(gi >= 2)
                def _deferred_drain():
                    drain16()

                for l in range(L):
                    @pl.when(l < rem)
                    def _issue():
                        col = colv[l]
                        b = bv0[l]
                        colb = jnp.broadcast_to(col, (L,))
                        for c0 in range(0, OUT, L):
                            ring_v[pl.ds(bank + l * OUT + c0, L)] = (
                                plsc.load_gather(buf3, [parv, cofs[c0 // L],
                                                        colb]))
                        pltpu.async_copy(
                            ring_v.at[pl.ds(bank + l * OUT, OUT)],
                            out_hbm.at[pl.ds(b * OUT, OUT)], sem)

                    @pl.when(l >= rem)
                    def _dummy():
                        pltpu.async_copy(idx_hbm.at[pl.ds(l * OUT, OUT)],
                                         drain_v.at[pl.ds(l * OUT, OUT)], sem)
                return gi + 1

            return lax.fori_loop(0, (kw + L - 1) // L, grp, gidx)

        def wloop(lw, gidx):
            par = lw & 1
            g = wid + lw * NW
            base = g * W

            @pl.when(g < nfull)
            def _wait_stream():
                pltpu.make_async_copy(wt_hbm.at[:, pl.ds(0, W)],
                                      buf3.at[par], sem_w).wait()

            @pl.when(g == nfull)
            def _tail_load():
                if item_station:
                    pltpu.sync_copy(wt_hbm.at[:, pl.ds(jnp.int32(PAD_I - 128),
                                                       128)],
                                    buf3.at[par, :, pl.ds(0, 128)])
                    pltpu.sync_copy(pad_hbm, buf3.at[par, :, pl.ds(128, 128)])
                else:
                    pltpu.sync_copy(pad_hbm, buf3.at[par, :, pl.ds(0, 128)])

            @pl.when(lw + 1 < n_me)
            def _prefetch():
                gn = g + NW

                @pl.when(gn < nfull)
                def _():
                    pltpu.async_copy(wt_hbm.at[:, pl.ds(gn * W, W)],
                                     buf3.at[1 - par], sem_w)

            return process_segment(par, base, lw, gidx)

        gidx = lax.fori_loop(0, n_me, wloop, jnp.int32(0))

        @pl.when(gidx >= 1)
        def _d1():
            drain16()

        @pl.when(gidx >= 2)
        def _d2():
            drain16()

    run_table(uid_hbm, wut_hbm, upad_hbm, ou_hbm, NFULL_U, False)
    run_table(iid_hbm, wit_hbm, ipad_hbm, oi_hbm, NFULL_I, True)

    for j in range(NCTX):
        pltpu.make_async_copy(oc_hbm.at[pl.ds(0, CBUF)], cb_v, sem_c).wait()


@jax.jit
def kernel(user_ids, item_ids, context_ids, W_user, W_item):
    del context_ids
    upad = jnp.pad(W_user[PAD_U:].T, ((0, 0), (0, 128 - (VU - PAD_U))))
    ipad = jnp.pad(W_item[PAD_I:].T, ((0, 0), (0, 128 - (VI - PAD_I))))
    u_flat, i_flat, c_flat = _sc_encode(
        user_ids.astype(jnp.int32), item_ids.astype(jnp.int32),
        jnp.transpose(W_user), jnp.transpose(W_item), upad, ipad)
    user_emb = u_flat.reshape(B, OUT)
    item_emb = i_flat.reshape(B, OUT)
    ctx_emb = c_flat.reshape(OUT, B).T
    return user_emb, item_emb, ctx_emb


# speedup vs baseline: 1.0015x; 1.0015x over previous
import functools

import jax
import jax.numpy as jnp
from jax import lax
from jax.experimental import pallas as pl
from jax.experimental.pallas import tpu as pltpu
from jax.experimental.pallas import tpu_sc as plsc

OUT = 64
B = 16384
CONST_VAL = 0.1
VU = 1000000
VI = 100000

NC = 2
NS = 16
L = 16
NW = NC * NS

W = 512
NFULL_U = VU // W
NFULL_I = VI // W
PAD_U = 999936
PAD_I = 99968
NBIN = 64
DUMP = 63
BSORT = B + NBIN * L
CBUF = 2048
NCTX = B * OUT // NW // CBUF

_mesh = plsc.VectorSubcoreMesh(core_axis_name="c", subcore_axis_name="s")


@functools.partial(
    pl.kernel,
    mesh=_mesh,
    compiler_params=pltpu.CompilerParams(
        use_tc_tiling_on_sc=True, needs_layout_passes=False),
    out_type=(
        jax.ShapeDtypeStruct((B * OUT,), jnp.float32),
        jax.ShapeDtypeStruct((B * OUT,), jnp.float32),
        jax.ShapeDtypeStruct((B * OUT,), jnp.float32),
    ),
    scratch_types=[
        pltpu.VMEM((B,), jnp.int32),
        pltpu.VMEM((BSORT,), jnp.int32),
        pltpu.VMEM((80,), jnp.int32),
        pltpu.VMEM((80,), jnp.int32),
        pltpu.VMEM((80,), jnp.int32),
        pltpu.VMEM((2, OUT, W), jnp.float32),
        pltpu.VMEM((2 * L * OUT,), jnp.float32),
        pltpu.VMEM((L * OUT,), jnp.int32),
        pltpu.VMEM((CBUF,), jnp.float32),
        pltpu.SemaphoreType.DMA,
        pltpu.SemaphoreType.DMA,
        pltpu.SemaphoreType.DMA,
    ],
)
def _sc_encode(uid_hbm, iid_hbm, wut_hbm, wit_hbm, upad_hbm, ipad_hbm,
               ou_hbm, oi_hbm, oc_hbm,
               ids_v, bsort_v, hist_v, seg_v, nxt_v, buf3, ring_v, drain_v,
               cb_v, sem, sem_w, sem_c):
    wid = lax.axis_index("s") * NC + lax.axis_index("c")
    lanes = lax.iota(jnp.int32, L)
    ones = jnp.ones((L,), jnp.int32)
    zeros = jnp.zeros((L,), jnp.int32)

    cvec = jnp.full((L,), CONST_VAL, jnp.float32)

    def cfill(i, _):
        cb_v[pl.ds(i * L, L)] = cvec
        return ()

    lax.fori_loop(0, CBUF // L, cfill, (), unroll=8)
    base_c = wid * (B * OUT // NW)
    for j in range(NCTX):
        pltpu.async_copy(cb_v, oc_hbm.at[pl.ds(base_c + j * CBUF, CBUF)],
                         sem_c)

    def run_table(idx_hbm, wt_hbm, pad_hbm, out_hbm, nfull, item_station):
        ntot = nfull + 1
        n_me = (ntot + (NW - 1) - wid) // NW

        with jax.named_scope("stage_ids"):
            pltpu.sync_copy(idx_hbm, ids_v)

        @pl.when(n_me > 0)
        def _prologue():
            pltpu.async_copy(wt_hbm.at[:, pl.ds(wid * W, W)], buf3.at[0],
                             sem_w)

        for c in range(5):
            hist_v[pl.ds(c * L, L)] = zeros

        def p1(i, _):
            idv = ids_v[pl.ds(i * L, L)]
            g = idv // W
            lgv = jnp.where((g & (NW - 1)) == wid, g >> 5, DUMP)
            plsc.addupdate_scatter(hist_v, [lgv], ones)
            return ()

        with jax.named_scope("pass1_hist"):
            lax.fori_loop(0, B // L, p1, (), unroll=4)

        carry = jnp.int32(0)
        for c in range(5):
            h = hist_v[pl.ds(c * L, L)]
            hp = (h + (L - 1)) & ~(L - 1)
            ps = plsc.cumsum(hp)
            seg = ps - hp + carry
            seg_v[pl.ds(c * L, L)] = seg
            nxt_v[pl.ds(c * L, L)] = seg
            carry = carry + ps[L - 1]

        def p3(i, _):
            idv = ids_v[pl.ds(i * L, L)]
            bv = lanes + i * L
            g = idv // W
            lgv = jnp.where((g & (NW - 1)) == wid, g >> 5, DUMP)
            nxtv = plsc.load_gather(nxt_v, [lgv])
            rank, last = plsc.scan_count(lgv)
            plsc.store_scatter(bsort_v, [nxtv + rank - 1], bv)
            plsc.store_scatter(nxt_v, [jnp.where(last, lgv, NBIN)],
                               nxtv + rank)
            return ()

        with jax.named_scope("pass3_scatter"):
            lax.fori_loop(0, B // L, p3, (), unroll=4)

        cofs = [jnp.broadcast_to(jnp.int32(c0), (L,)) + lanes
                for c0 in range(0, OUT, L)]

        def drain16():
            pltpu.make_async_copy(idx_hbm.at[pl.ds(0, L * OUT)], drain_v,
                                  sem).wait()

        def process_segment(par, base, lg, gidx):
            parv = jnp.broadcast_to(par, (L,))
            cb = (lg >> 4) << 4
            lane = lg & (L - 1)
            hch = hist_v[pl.ds(cb, L)]
            sch = seg_v[pl.ds(cb, L)]
            kw = jnp.max(jnp.where(lanes == lane, hch, 0))
            s = jnp.max(jnp.where(lanes == lane, sch, 0))

            def grp(t16, gi):
                bv0 = bsort_v[pl.ds(s + t16 * L, L)]
                rem = kw - t16 * L
                bvc = jnp.where(lanes < rem, bv0, 0)
                colv = jnp.where(lanes < rem,
                                 plsc.load_gather(ids_v, [bvc]) - base, 0)
                bank = (gi & 1) * (L * OUT)

                @pl.when(gi >= 2)
                def _deferred_drain():
                    drain16()

                for l in range(L):
                    @pl.when(l < rem)
                    def _issue():
                        col = colv[l]
                        b = bv0[l]
                        colb = jnp.broadcast_to(col, (L,))
                        for c0 in range(0, OUT, L):
                            ring_v[pl.ds(bank + l * OUT + c0, L)] = (
                                plsc.load_gather(buf3, [parv, cofs[c0 // L],
                                                        colb]))
                        pltpu.async_copy(
                            ring_v.at[pl.ds(bank + l * OUT, OUT)],
                            out_hbm.at[pl.ds(b * OUT, OUT)], sem)

                    @pl.when(l >= rem)
                    def _dummy():
                        pltpu.async_copy(idx_hbm.at[pl.ds(l * OUT, OUT)],
                                         drain_v.at[pl.ds(l * OUT, OUT)], sem)
                return gi + 1

            return lax.fori_loop(0, (kw + L - 1) // L, grp, gidx)

        def wloop(lw, gidx):
            par = lw & 1
            g = wid + lw * NW
            base = g * W

            @pl.when(g < nfull)
            def _wait_stream():
                pltpu.make_async_copy(wt_hbm.at[:, pl.ds(0, W)],
                                      buf3.at[par], sem_w).wait()

            @pl.when(g == nfull)
            def _tail_load():
                if item_station:
                    pltpu.sync_copy(wt_hbm.at[:, pl.ds(jnp.int32(PAD_I - 128),
                                                       128)],
                                    buf3.at[par, :, pl.ds(0, 128)])
                    pltpu.sync_copy(pad_hbm, buf3.at[par, :, pl.ds(128, 128)])
                else:
                    pltpu.sync_copy(pad_hbm, buf3.at[par, :, pl.ds(0, 128)])

            @pl.when(lw + 1 < n_me)
            def _prefetch():
                gn = g + NW

                @pl.when(gn < nfull)
                def _():
                    pltpu.async_copy(wt_hbm.at[:, pl.ds(gn * W, W)],
                                     buf3.at[1 - par], sem_w)

            return process_segment(par, base, lw, gidx)

        with jax.named_scope("window_loop"):
            gidx = lax.fori_loop(0, n_me, wloop, jnp.int32(0))

        @pl.when(gidx >= 1)
        def _d1():
            drain16()

        @pl.when(gidx >= 2)
        def _d2():
            drain16()

    run_table(uid_hbm, wut_hbm, upad_hbm, ou_hbm, NFULL_U, False)
    run_table(iid_hbm, wit_hbm, ipad_hbm, oi_hbm, NFULL_I, True)

    for j in range(NCTX):
        pltpu.make_async_copy(oc_hbm.at[pl.ds(0, CBUF)], cb_v, sem_c).wait()


@jax.jit
def kernel(user_ids, item_ids, context_ids, W_user, W_item):
    del context_ids
    upad = jnp.pad(W_user[PAD_U:].T, ((0, 0), (0, 128 - (VU - PAD_U))))
    ipad = jnp.pad(W_item[PAD_I:].T, ((0, 0), (0, 128 - (VI - PAD_I))))
    u_flat, i_flat, c_flat = _sc_encode(
        user_ids.astype(jnp.int32), item_ids.astype(jnp.int32),
        jnp.transpose(W_user), jnp.transpose(W_item), upad, ipad)
    user_emb = u_flat.reshape(B, OUT)
    item_emb = i_flat.reshape(B, OUT)
    ctx_emb = c_flat.reshape(OUT, B).T
    return user_emb, item_emb, ctx_emb
